# baseline (speedup 1.0000x reference)
import jax
import jax.numpy as jnp
from jax import lax
from jax.experimental import pallas as pl
from jax.experimental.pallas import tpu as pltpu
from jax.experimental.pallas import tpu_sc as plsc

MAX_TILES = 4
LANES = 16


def _pos_body(ar_hbm, emb_hbm, pos_hbm, ar_v, idx_v, row_v):
    wid = lax.axis_index("s") * 2 + lax.axis_index("c")
    b = wid // MAX_TILES
    t = wid % MAX_TILES
    ar_v[pl.ds(LANES, LANES)] = jnp.full((LANES,), 1, jnp.int32)
    pltpu.sync_copy(ar_hbm, ar_v.at[pl.ds(0, LANES)])
    vh = ar_v[pl.ds(2 * b, LANES)]
    vw = ar_v[pl.ds(2 * b + 1, LANES)]
    rr = (jnp.where(t >= vw, 1, 0) + jnp.where(t >= 2 * vw, 1, 0)
          + jnp.where(t >= 3 * vw, 1, 0))
    cc = t - rr * vw
    fi = rr * MAX_TILES + cc
    fi = jnp.where(t < vh * vw, fi, MAX_TILES * MAX_TILES)
    idx_v[...] = fi
    pltpu.sync_copy(emb_hbm.at[idx_v.at[pl.ds(0, 1)]], row_v)
    pltpu.sync_copy(row_v, pos_hbm.at[pl.ds(wid, 1)])


def _masked_pos(ar_flat, emb_flat):
    d = emb_flat.shape[1]
    mesh = plsc.VectorSubcoreMesh(core_axis_name="c", subcore_axis_name="s")
    return pl.kernel(
        _pos_body,
        out_type=jax.ShapeDtypeStruct((2 * LANES, d), jnp.float32),
        mesh=mesh,
        scratch_types=[
            pltpu.VMEM((2 * LANES,), jnp.int32),
            pltpu.VMEM((LANES,), jnp.int32),
            pltpu.VMEM((1, d), jnp.float32),
        ],
    )(ar_flat, emb_flat)


TOK_BLK = 544


def _add_body(gate_ref, x_ref, pos_ref, o_ref):
    g = jnp.tanh(gate_ref[0])
    o_ref[...] = x_ref[...] + g * pos_ref[...]


def _broadcast_add(gate, xt, pos_r):
    bsz, n, n_tiles, d = xt.shape
    nb = pl.cdiv(n, TOK_BLK)
    return pl.pallas_call(
        _add_body,
        grid=(bsz, nb),
        in_specs=[
            pl.BlockSpec(memory_space=pltpu.SMEM),
            pl.BlockSpec((1, TOK_BLK, n_tiles, d), lambda i, k: (i, k, 0, 0)),
            pl.BlockSpec((1, 1, n_tiles, d), lambda i, k: (i, 0, 0, 0)),
        ],
        out_specs=pl.BlockSpec((1, TOK_BLK, n_tiles, d),
                               lambda i, k: (i, k, 0, 0)),
        out_shape=jax.ShapeDtypeStruct((bsz, n, n_tiles, d), jnp.float32),
    )(gate, xt, pos_r)


def kernel(x, aspect_ratio, embedding, gate):
    bsz, n_tiles, n_tok, d = x.shape
    ar_flat = aspect_ratio.astype(jnp.int32).reshape(-1)
    emb_flat = jnp.concatenate(
        [embedding.reshape(MAX_TILES * MAX_TILES, d),
         jnp.zeros((1, d), jnp.float32)], axis=0)
    pos = _masked_pos(ar_flat, emb_flat)
    xt = jnp.transpose(x, (0, 2, 1, 3))
    pos_r = pos.reshape(bsz, 1, n_tiles, d)
    outt = _broadcast_add(gate, xt, pos_r)
    return jnp.transpose(outt, (0, 2, 1, 3))


# speedup vs baseline: 1.0046x; 1.0046x over previous
import jax
import jax.numpy as jnp
from jax import lax
from jax.experimental import pallas as pl
from jax.experimental.pallas import tpu as pltpu
from jax.experimental.pallas import tpu_sc as plsc

MAX_TILES = 4
LANES = 16


def _pos_body(ar_hbm, emb_hbm, pos_hbm, ar_v, idx_v, row_v):
    wid = lax.axis_index("s") * 2 + lax.axis_index("c")
    b = wid // MAX_TILES
    t = wid % MAX_TILES
    ar_v[pl.ds(LANES, LANES)] = jnp.full((LANES,), 1, jnp.int32)
    pltpu.sync_copy(ar_hbm, ar_v.at[pl.ds(0, LANES)])
    vh = ar_v[pl.ds(2 * b, LANES)]
    vw = ar_v[pl.ds(2 * b + 1, LANES)]
    rr = (jnp.where(t >= vw, 1, 0) + jnp.where(t >= 2 * vw, 1, 0)
          + jnp.where(t >= 3 * vw, 1, 0))
    cc = t - rr * vw
    fi = rr * MAX_TILES + cc
    fi = jnp.where(t < vh * vw, fi, MAX_TILES * MAX_TILES)
    idx_v[...] = fi
    pltpu.sync_copy(emb_hbm.at[idx_v.at[pl.ds(0, 1)]], row_v)
    pltpu.sync_copy(row_v, pos_hbm.at[pl.ds(wid, 1)])


def _masked_pos(ar_flat, emb_flat):
    d = emb_flat.shape[1]
    mesh = plsc.VectorSubcoreMesh(core_axis_name="c", subcore_axis_name="s")
    return pl.kernel(
        _pos_body,
        out_type=jax.ShapeDtypeStruct((2 * LANES, d), jnp.float32),
        mesh=mesh,
        scratch_types=[
            pltpu.VMEM((2 * LANES,), jnp.int32),
            pltpu.VMEM((LANES,), jnp.int32),
            pltpu.VMEM((1, d), jnp.float32),
        ],
    )(ar_flat, emb_flat)


TOK_BLK = 512


def _add_body(gate_ref, x_ref, pos_ref, o_ref):
    g = jnp.tanh(gate_ref[0])
    o_ref[...] = x_ref[...] + g * pos_ref[...]


def _broadcast_add(gate, xt, pos_r):
    bsz, n, n_tiles, d = xt.shape
    nb = pl.cdiv(n, TOK_BLK)
    return pl.pallas_call(
        _add_body,
        grid=(bsz, nb),
        in_specs=[
            pl.BlockSpec(memory_space=pltpu.SMEM),
            pl.BlockSpec((1, TOK_BLK, n_tiles, d), lambda i, k: (i, k, 0, 0)),
            pl.BlockSpec((1, 1, n_tiles, d), lambda i, k: (i, 0, 0, 0)),
        ],
        out_specs=pl.BlockSpec((1, TOK_BLK, n_tiles, d),
                               lambda i, k: (i, k, 0, 0)),
        out_shape=jax.ShapeDtypeStruct((bsz, n, n_tiles, d), jnp.float32),
    )(gate, xt, pos_r)


def kernel(x, aspect_ratio, embedding, gate):
    bsz, n_tiles, n_tok, d = x.shape
    ar_flat = aspect_ratio.astype(jnp.int32).reshape(-1)
    emb_flat = jnp.concatenate(
        [embedding.reshape(MAX_TILES * MAX_TILES, d),
         jnp.zeros((1, d), jnp.float32)], axis=0)
    pos = _masked_pos(ar_flat, emb_flat)
    xt = jnp.transpose(x, (0, 2, 1, 3))
    pos_r = pos.reshape(bsz, 1, n_tiles, d)
    outt = _broadcast_add(gate, xt, pos_r)
    return jnp.transpose(outt, (0, 2, 1, 3))
